# Initial kernel scaffold; baseline (speedup 1.0000x reference)
import jax
import jax.numpy as jnp
from jax.experimental import pallas as pl


def kernel(logits, labels):
    raise NotImplementedError("write your pallas kernel here")


# speedup vs baseline: 12.8683x; 12.8683x over previous
import jax
import jax.numpy as jnp
from jax import lax
from jax.experimental import pallas as pl
from jax.experimental.pallas import tpu as pltpu
from jax.experimental.pallas import tpu_sc as plsc

N = 16 * 512 * 512
BKT = 1 << 18
SHIFT = 13
HIST = 2 * BKT
NC, NS = 2, 16
NW = NC * NS
ELEMS = N // NW
CHUNK = 8192
NCHUNK = ELEMS // CHUNK
WORDS = HIST // NS
BB = 8192


def _hist_body(scores_hbm, labels_hbm, cnt_out, rsum_out,
               s_v, l_v, idx_v, relu_v, ones_v, buf_v, cnt_sh, rsum_sh):
    cid = lax.axis_index("c")
    sid = lax.axis_index("s")
    wid = sid * NC + cid

    def fill(i, _):
        o = i * 16
        buf_v[pl.ds(o, 16)] = jnp.zeros((16,), jnp.float32)
        ones_v[pl.ds(o, 16)] = jnp.ones((16,), jnp.float32)
        return 0
    lax.fori_loop(0, CHUNK // 16, fill, 0)

    def zero_slices(t, _):
        off = sid * WORDS + t * BB
        pltpu.sync_copy(buf_v, cnt_sh.at[pl.ds(off, BB)])
        pltpu.sync_copy(buf_v, rsum_sh.at[pl.ds(off, BB)])
        return 0
    lax.fori_loop(0, WORDS // BB, zero_slices, 0)
    plsc.subcore_barrier()

    def chunk(k, _):
        start = wid * ELEMS + k * CHUNK
        pltpu.sync_copy(scores_hbm.at[pl.ds(start, CHUNK)], s_v)
        pltpu.sync_copy(labels_hbm.at[pl.ds(start, CHUNK)], l_v)

        def elem(i, _):
            o = i * 16
            s = s_v[pl.ds(o, 16)]
            l = l_v[pl.ds(o, 16)]
            lf = l.astype(jnp.float32)
            e = 1.0 - s * (2.0 * lf - 1.0)
            relu_v[pl.ds(o, 16)] = jnp.maximum(e, 0.0)
            bits = lax.bitcast_convert_type(e, jnp.int32)
            bid = jnp.where(e > 0.0, lax.shift_right_logical(bits, SHIFT), 0)
            idx_v[pl.ds(o, 16)] = bid + l * BKT
            return 0
        lax.fori_loop(0, CHUNK // 16, elem, 0)

        pltpu.sync_copy(ones_v, cnt_sh.at[idx_v], add=True)
        pltpu.sync_copy(relu_v, rsum_sh.at[idx_v], add=True)
        return 0
    lax.fori_loop(0, NCHUNK, chunk, 0)
    plsc.subcore_barrier()

    def wout(t, _):
        off = sid * WORDS + t * BB
        pltpu.sync_copy(cnt_sh.at[pl.ds(off, BB)], buf_v)
        pltpu.sync_copy(buf_v, cnt_out.at[cid, pl.ds(off, BB)])
        pltpu.sync_copy(rsum_sh.at[pl.ds(off, BB)], buf_v)
        pltpu.sync_copy(buf_v, rsum_out.at[cid, pl.ds(off, BB)])
        return 0
    lax.fori_loop(0, WORDS // BB, wout, 0)


def _finalize_body(cnt_ref, rsum_ref, out_ref):
    cn = cnt_ref[0] + cnt_ref[2]
    cp = cnt_ref[1] + cnt_ref[3]
    rs = rsum_ref[0] + rsum_ref[1] + rsum_ref[2] + rsum_ref[3]
    cnt = cn + cp
    P = jnp.sum(cp)

    iu = lax.broadcasted_iota(jnp.int32, (512, 512), 0)
    il = lax.broadcasted_iota(jnp.int32, (512, 512), 1)
    U = (iu > il).astype(jnp.float32)

    sk = lax.dot(cnt, U)
    sp = lax.dot(cp, U)
    rowk = jnp.sum(cnt, axis=1, keepdims=True)
    rowp = jnp.sum(cp, axis=1, keepdims=True)
    W = (il > iu).astype(jnp.float32)
    k1 = sk + lax.dot(W, rowk)
    p1 = sp + lax.dot(W, rowp)

    n1 = k1 - p1
    n2 = n1 + cn
    num = (P - p1) * cn + cp * (P + n1)
    den = (P + n1) * (P + n2)
    dJ = jnp.where(den > 0.0, num / jnp.where(den > 0.0, den, 1.0), 0.0)
    dJ0 = ((k1 == 0.0) & (cnt > 0.0)).astype(jnp.float32)
    dJ = jnp.where(P > 0.0, dJ, dJ0)
    rbar = jnp.where(cnt > 0.0, rs / jnp.where(cnt > 0.0, cnt, 1.0), 0.0)
    out_ref[...] = jnp.broadcast_to(jnp.sum(rbar * dJ), (1, 1))


def kernel(logits, labels):
    scores = logits.reshape(-1).astype(jnp.float32)
    lab = labels.reshape(-1).astype(jnp.int32)

    mesh = plsc.VectorSubcoreMesh(core_axis_name="c", subcore_axis_name="s")
    hist = pl.kernel(
        _hist_body,
        mesh=mesh,
        out_type=[
            jax.ShapeDtypeStruct((NC, HIST), jnp.float32),
            jax.ShapeDtypeStruct((NC, HIST), jnp.float32),
        ],
        scratch_types=[
            pltpu.VMEM((CHUNK,), jnp.float32),
            pltpu.VMEM((CHUNK,), jnp.int32),
            pltpu.VMEM((CHUNK,), jnp.int32),
            pltpu.VMEM((CHUNK,), jnp.float32),
            pltpu.VMEM((CHUNK,), jnp.float32),
            pltpu.VMEM((BB,), jnp.float32),
            pltpu.VMEM_SHARED((HIST,), jnp.float32),
            pltpu.VMEM_SHARED((HIST,), jnp.float32),
        ],
    )
    cnt2, rsum2 = hist(scores, lab)

    cnt4 = cnt2.reshape(4, 512, 512)
    rsum4 = rsum2.reshape(4, 512, 512)
    loss = pl.pallas_call(
        _finalize_body,
        out_shape=jax.ShapeDtypeStruct((1, 1), jnp.float32),
    )(cnt4, rsum4)
    return loss.reshape(())
